# Initial kernel scaffold; baseline (speedup 1.0000x reference)
import jax
import jax.numpy as jnp
from jax.experimental import pallas as pl


def kernel(x, pos, edge_index, W_emb, Wq, Wk, Wv, Wo, Ww):
    raise NotImplementedError("write your pallas kernel here")


# speedup vs baseline: 2.5619x; 2.5619x over previous
import functools

import jax
import jax.numpy as jnp
import numpy as np
from jax import lax
from jax.experimental import pallas as pl
from jax.experimental.pallas import tpu as pltpu
from jax.experimental.pallas import tpu_sc as plsc

N, E, D, D_IN, NB, L = 50000, 800000, 86, 3, 10, 3
MAX_RADIUS = 2.0
SCALE = 1.0 / np.sqrt(D)

DP = 96
NRB = 16
NPB = 256
NBKT = 196
BKT_PAD = 208
NPAD = NBKT * NPB
NWRK = 32
EWP = 25088
EIN = NWRK * EWP
EPAD = 802816
EDW = EPAD // NWRK
CH = 128
RB = 1792

_mesh = plsc.VectorSubcoreMesh(core_axis_name="c", subcore_axis_name="s")
_f32 = jnp.float32
_i32 = jnp.int32


def _wid():
    return lax.axis_index("s") * 2 + lax.axis_index("c")


def _iota16():
    return lax.broadcasted_iota(_i32, (16,), 0)


def _full(v, dtype=_i32):
    return jnp.full((16,), v, dtype)


def _elem(ref, i):
    g = plsc.load_gather(ref, [_full(i)])
    return jnp.sum(jnp.where(_iota16() == 0, g, jnp.zeros_like(g)))


@functools.partial(
    pl.kernel,
    compiler_params=pltpu.CompilerParams(needs_layout_passes=False, use_tc_tiling_on_sc=False),
    out_type=jax.ShapeDtypeStruct((NWRK, BKT_PAD * 16), _i32),
    mesh=_mesh,
    scratch_types=[
        pltpu.VMEM((EWP,), _i32),
        pltpu.VMEM((BKT_PAD * 16,), _i32),
    ],
)
def _k_hist(dst_hbm, cnt_hbm, dst_v, cnt_v):
    w = _wid()
    pltpu.sync_copy(dst_hbm.at[pl.ds(pl.multiple_of(w * EWP, 128), EWP)], dst_v)
    zeros = jnp.zeros((16,), _i32)
    it16 = _iota16()

    def zb(i, _):
        cnt_v[pl.ds(i * 16, 16)] = zeros
        return 0

    lax.fori_loop(0, BKT_PAD, zb, 0)

    ebase = w * EWP

    def body(g, _):
        d16 = dst_v[pl.ds(g * 16, 16)]
        msk = (ebase + g * 16 + it16) < E
        idx = (d16 >> 8) * 16 + it16
        c = plsc.load_gather(cnt_v, [idx], mask=msk)
        plsc.store_scatter(cnt_v, [idx], c + 1, mask=msk)
        return 0

    lax.fori_loop(0, EWP // 16, body, 0)
    pltpu.sync_copy(cnt_v, cnt_hbm.at[w])


@functools.partial(
    pl.kernel,
    compiler_params=pltpu.CompilerParams(needs_layout_passes=False, use_tc_tiling_on_sc=False),
    out_type=(
        jax.ShapeDtypeStruct((EPAD,), _i32),
        jax.ShapeDtypeStruct((EPAD,), _i32),
        jax.ShapeDtypeStruct((BKT_PAD,), _i32),
        jax.ShapeDtypeStruct((BKT_PAD,), _i32),
    ),
    mesh=_mesh,
    scratch_types=[
        pltpu.VMEM((NWRK * BKT_PAD * 16,), _i32),
        pltpu.VMEM((BKT_PAD * 16,), _i32),
        pltpu.VMEM((BKT_PAD,), _i32),
        pltpu.VMEM((BKT_PAD,), _i32),
        pltpu.VMEM((CH,), _i32),
        pltpu.VMEM((CH,), _i32),
        pltpu.VMEM((CH,), _i32),
        pltpu.SemaphoreType.DMA,
    ],
)
def _k_place(dst_hbm, src_hbm, cntf_hbm, srcp_hbm, dstp_hbm, offp_hbm, deg_hbm,
             cnt_a, base16, deg_v, offp_v, srcv, dstv, posv, sem):
    w = _wid()
    pltpu.sync_copy(cntf_hbm, cnt_a)
    it16 = _iota16()
    zeros = jnp.zeros((16,), _i32)

    def degb(b, _):
        def accw(wi, s):
            return s + cnt_a[pl.ds((wi * BKT_PAD + b) * 16, 16)]

        tot = lax.fori_loop(0, NWRK, accw, zeros)
        s = jnp.sum(tot)
        plsc.store_scatter(deg_v, [_full(b)], _full(s), mask=it16 == 0)
        return 0

    lax.fori_loop(0, BKT_PAD, degb, 0)

    def pfx(gi, carry):
        d16 = deg_v[pl.ds(gi * 16, 16)]
        r16 = (d16 + 7) & _full(-8)
        cs = plsc.cumsum(r16)
        offp_v[pl.ds(gi * 16, 16)] = carry + cs - r16
        return carry + jnp.sum(r16)

    lax.fori_loop(0, BKT_PAD // 16, pfx, jnp.int32(0))

    @pl.when(w == 0)
    def _():
        pltpu.sync_copy(offp_v, offp_hbm)
        pltpu.sync_copy(deg_v, deg_hbm)

    def baseb(b, _):
        def accw(wi, s):
            row = cnt_a[pl.ds((wi * BKT_PAD + b) * 16, 16)]
            return s + jnp.where(wi < w, jnp.sum(row), 0)

        before = lax.fori_loop(0, NWRK, accw, jnp.int32(0))
        myrow = cnt_a[pl.ds((w * BKT_PAD + b) * 16, 16)]
        mycs = plsc.cumsum(myrow) - myrow
        ob = plsc.load_gather(offp_v, [_full(b)])
        base16[pl.ds(b * 16, 16)] = ob + before + mycs
        return 0

    lax.fori_loop(0, BKT_PAD, baseb, 0)

    dump = _full(EPAD - 16) + it16

    def chunk(ci, _):
        cb = pl.multiple_of(w * EWP + ci * CH, 128)
        pltpu.sync_copy(dst_hbm.at[pl.ds(cb, CH)], dstv)
        pltpu.sync_copy(src_hbm.at[pl.ds(cb, CH)], srcv)

        def place(g, _):
            d16 = dstv[pl.ds(g * 16, 16)]
            msk = (cb + g * 16 + it16) < E
            idx = (d16 >> 8) * 16 + it16
            p = plsc.load_gather(base16, [idx], mask=msk)
            plsc.store_scatter(base16, [idx], p + 1, mask=msk)
            posv[pl.ds(g * 16, 16)] = jnp.where(msk, p, dump)
            return 0

        lax.fori_loop(0, CH // 16, place, 0)
        pltpu.async_copy(srcv, srcp_hbm.at[posv], sem).wait()
        pltpu.async_copy(dstv, dstp_hbm.at[posv], sem).wait()
        return 0

    lax.fori_loop(0, EWP // CH, chunk, 0)


@functools.partial(
    pl.kernel,
    compiler_params=pltpu.CompilerParams(needs_layout_passes=False, use_tc_tiling_on_sc=False),
    out_type=jax.ShapeDtypeStruct((EPAD,), _f32),
    mesh=_mesh,
    scratch_types=[
        pltpu.VMEM((CH,), _i32),
        pltpu.VMEM((CH,), _i32),
        pltpu.VMEM((CH, 16), _f32),
        pltpu.VMEM((CH, 16), _f32),
        pltpu.VMEM((CH,), _f32),
        pltpu.SemaphoreType.DMA,
        pltpu.SemaphoreType.DMA,
    ],
)
def _k_dist(srcp_hbm, dstp_hbm, pos_hbm, dist_hbm, sv, dv, ps, pd, dout,
            sem1, sem2):
    w = _wid()
    it16 = _iota16()
    nmax = _full(N - 1)
    zeroi = jnp.zeros((16,), _i32)
    magic = _full(0x1FBD1DF5)
    half = _full(0.5, _f32)
    eps = _full(1e-12, _f32)

    def chunk(ci, _):
        e0 = pl.multiple_of(w * EDW + ci * CH, 128)
        pltpu.sync_copy(srcp_hbm.at[pl.ds(e0, CH)], sv)
        pltpu.sync_copy(dstp_hbm.at[pl.ds(e0, CH)], dv)

        def clampg(g, _):
            sv[pl.ds(g * 16, 16)] = jnp.minimum(
                jnp.maximum(sv[pl.ds(g * 16, 16)], zeroi), nmax)
            dv[pl.ds(g * 16, 16)] = jnp.minimum(
                jnp.maximum(dv[pl.ds(g * 16, 16)], zeroi), nmax)
            return 0

        lax.fori_loop(0, CH // 16, clampg, 0)
        pltpu.async_copy(pos_hbm.at[sv], ps, sem1).wait()
        pltpu.async_copy(pos_hbm.at[dv], pd, sem2).wait()

        def dot3(g, _):
            rows = g * 16 + it16
            s = eps
            for cdim in range(3):
                a = plsc.load_gather(ps, [rows, _full(cdim)])
                b = plsc.load_gather(pd, [rows, _full(cdim)])
                d = a - b
                s = s + d * d
            i = plsc.bitcast(s, _i32)
            y = plsc.bitcast(magic + (i >> 1), _f32)
            for _ in range(3):
                y = half * (y + s / y)
            dout[pl.ds(g * 16, 16)] = y
            return 0

        lax.fori_loop(0, CH // 16, dot3, 0)
        pltpu.sync_copy(dout, dist_hbm.at[pl.ds(e0, CH)])
        return 0

    lax.fori_loop(0, EDW // CH, chunk, 0)


_RBF_C = np.linspace(0.0, MAX_RADIUS, NB)
_RBF_I = 1.0 / (2.0 * (MAX_RADIUS / NB) ** 2)


@functools.partial(
    pl.kernel,
    compiler_params=pltpu.CompilerParams(needs_layout_passes=False, use_tc_tiling_on_sc=False),
    out_type=(
        jax.ShapeDtypeStruct((NPAD, DP), _f32),
        jax.ShapeDtypeStruct((NPAD, NRB), _f32),
    ),
    mesh=_mesh,
    scratch_types=[
        pltpu.VMEM((NPB, DP), _f32),
        pltpu.VMEM((NPB, NRB), _f32),
        pltpu.VMEM((CH, DP), _f32),
        pltpu.VMEM((CH, DP), _f32),
        pltpu.VMEM((CH, NRB), _f32),
        pltpu.VMEM((CH,), _i32),
        pltpu.VMEM((CH,), _i32),
        pltpu.VMEM((CH,), _f32),
        pltpu.VMEM((CH,), _i32),
        pltpu.VMEM((CH,), _i32),
        pltpu.VMEM((CH,), _i32),
        pltpu.VMEM((BKT_PAD,), _i32),
        pltpu.VMEM((BKT_PAD,), _i32),
        pltpu.VMEM((CH, DP), _f32),
        pltpu.VMEM_SHARED((16 * NPB, DP), _f32),
        pltpu.VMEM_SHARED((16 * NPB, NRB), _f32),
        pltpu.SemaphoreType.DMA,
        pltpu.SemaphoreType.DMA,
        pltpu.SemaphoreType.DMA,
        pltpu.SemaphoreType.DMA,
    ],
)
def _k_sweep(q_hbm, qr_hbm, kt_hbm, vt_hbm, srcp_hbm, dstp_hbm, dist_hbm,
             offp_hbm, deg_hbm, numv_hbm, numrd_hbm,
             q_l, qr_l, kc, vc, urd, srcv, dstv, distv, sidx, aidx, dlv,
             offp_v, deg_v, zblk, accv, accrd, semk, semv, sema, semb):
    w = _wid()
    sid = lax.axis_index("s")
    it16 = _iota16()
    pltpu.sync_copy(offp_hbm, offp_v)
    pltpu.sync_copy(deg_hbm, deg_v)

    zeros = jnp.zeros((16,), _f32)
    zeroi = jnp.zeros((16,), _i32)
    nmax = _full(N - 1)
    npbm1 = _full(NPB - 1)

    def zrow(r, _):
        for c in range(DP // 16):
            zblk[r, pl.ds(c * 16, 16)] = zeros
        return 0

    lax.fori_loop(0, CH, zrow, 0)

    for t in range(7):
        b = w + NWRK * t

        @pl.when(b < NBKT)
        def _():
            pltpu.sync_copy(q_hbm.at[pl.ds(pl.multiple_of(b * NPB, NPB), NPB)], q_l)
            pltpu.sync_copy(qr_hbm.at[pl.ds(pl.multiple_of(b * NPB, NPB), NPB)], qr_l)
            for z in range(NPB // CH):
                pltpu.sync_copy(zblk, accv.at[pl.ds(sid * NPB + z * CH, CH)])
                pltpu.sync_copy(zblk.at[:, pl.ds(0, NRB)],
                                accrd.at[pl.ds(sid * NPB + z * CH, CH)])

            degb = _elem(deg_v, b)
            offb = _elem(offp_v, b)
            nch = (degb + CH - 1) // CH

            def chunk(ci, _):
                e0 = pl.multiple_of(offb + ci * CH, 8)
                pltpu.sync_copy(srcp_hbm.at[pl.ds(e0, CH)], srcv)
                pltpu.sync_copy(dstp_hbm.at[pl.ds(e0, CH)], dstv)
                pltpu.sync_copy(dist_hbm.at[pl.ds(e0, CH)], distv)

                def cleang(g, _):
                    s16 = srcv[pl.ds(g * 16, 16)]
                    d16 = dstv[pl.ds(g * 16, 16)]
                    sidx[pl.ds(g * 16, 16)] = jnp.minimum(
                        jnp.maximum(s16, zeroi), nmax)
                    dl = jnp.minimum(jnp.maximum(d16 - b * NPB, zeroi), npbm1)
                    dlv[pl.ds(g * 16, 16)] = dl
                    aidx[pl.ds(g * 16, 16)] = dl + sid * NPB
                    return 0

                lax.fori_loop(0, CH // 16, cleang, 0)
                pltpu.async_copy(kt_hbm.at[sidx], kc, semk).wait()
                cpv = pltpu.async_copy(vt_hbm.at[sidx], vc, semv)

                def logitg(g, _):
                    dlg = dlv[pl.ds(g * 16, 16)]
                    rows = g * 16 + it16

                    def dotd(dd, acc):
                        qv = plsc.load_gather(q_l, [dlg, _full(dd)])
                        kv = plsc.load_gather(kc, [rows, _full(dd)])
                        return acc + qv * kv

                    acc = lax.fori_loop(0, D, dotd, zeros, unroll=2)

                    dg = distv[pl.ds(g * 16, 16)]
                    acc2 = zeros
                    rjs = []
                    for j in range(NB):
                        dd = dg - _full(_RBF_C[j], _f32)
                        rj = jnp.exp(dd * dd * _full(-_RBF_I, _f32))
                        qrv = plsc.load_gather(qr_l, [dlg, _full(j)])
                        acc2 = acc2 + qrv * rj
                        rjs.append(rj)
                    lg = acc + acc2
                    valid = (ci * CH + g * 16 + it16) < degb
                    ex = jnp.where(valid, jnp.exp(lg), zeros)
                    for j in range(NB):
                        plsc.store_scatter(urd, [rows, _full(j)], rjs[j] * ex)
                    plsc.store_scatter(urd, [rows, _full(NB)], ex)
                    for j in range(NB + 1, NRB):
                        plsc.store_scatter(urd, [rows, _full(j)], zeros)

                    def upd(dd, exc):
                        vv = plsc.load_gather(vc, [rows, _full(dd)])
                        plsc.store_scatter(vc, [rows, _full(dd)], vv * exc)
                        return exc

                    lax.fori_loop(0, D, upd, ex, unroll=2)
                    return 0

                cpv.wait()
                lax.fori_loop(0, CH // 16, logitg, 0)
                pltpu.async_copy(vc, accv.at[aidx], sema, add=True).wait()
                pltpu.async_copy(urd, accrd.at[aidx], semb, add=True).wait()
                return 0

            lax.fori_loop(0, nch, chunk, 0)
            pltpu.sync_copy(accv.at[pl.ds(sid * NPB, NPB)],
                            numv_hbm.at[pl.ds(pl.multiple_of(b * NPB, NPB), NPB)])
            pltpu.sync_copy(accrd.at[pl.ds(sid * NPB, NPB)],
                            numrd_hbm.at[pl.ds(pl.multiple_of(b * NPB, NPB), NPB)])


def _tc_embed(xp, wembp):
    def body(x_ref, w_ref, o_ref):
        o_ref[...] = jnp.dot(x_ref[...], w_ref[...],
                             preferred_element_type=_f32)

    return pl.pallas_call(
        body,
        grid=(NPAD // RB,),
        in_specs=[
            pl.BlockSpec((RB, 8), lambda i: (i, 0)),
            pl.BlockSpec((8, DP), lambda i: (0, 0)),
        ],
        out_specs=pl.BlockSpec((RB, DP), lambda i: (i, 0)),
        out_shape=jax.ShapeDtypeStruct((NPAD, DP), _f32),
    )(xp, wembp)


def _tc_pre(h, aq, akh, avh, aqrt):
    def body(h_ref, aq_ref, akh_ref, avh_ref, aqrt_ref,
             q_ref, qr_ref, kn_ref, vn_ref):
        hb = h_ref[...]
        q = jnp.dot(hb, aq_ref[...], preferred_element_type=_f32) * SCALE
        q_ref[...] = q
        qr_ref[...] = jnp.dot(q, aqrt_ref[...], preferred_element_type=_f32)
        kn_ref[...] = jnp.dot(hb, akh_ref[...], preferred_element_type=_f32)
        vn_ref[...] = jnp.dot(hb, avh_ref[...], preferred_element_type=_f32)

    return pl.pallas_call(
        body,
        grid=(NPAD // RB,),
        in_specs=[
            pl.BlockSpec((RB, DP), lambda i: (i, 0)),
            pl.BlockSpec((DP, DP), lambda i: (0, 0)),
            pl.BlockSpec((DP, DP), lambda i: (0, 0)),
            pl.BlockSpec((DP, DP), lambda i: (0, 0)),
            pl.BlockSpec((DP, NRB), lambda i: (0, 0)),
        ],
        out_specs=[
            pl.BlockSpec((RB, DP), lambda i: (i, 0)),
            pl.BlockSpec((RB, NRB), lambda i: (i, 0)),
            pl.BlockSpec((RB, DP), lambda i: (i, 0)),
            pl.BlockSpec((RB, DP), lambda i: (i, 0)),
        ],
        out_shape=(
            jax.ShapeDtypeStruct((NPAD, DP), _f32),
            jax.ShapeDtypeStruct((NPAD, NRB), _f32),
            jax.ShapeDtypeStruct((NPAD, DP), _f32),
            jax.ShapeDtypeStruct((NPAD, DP), _f32),
        ),
    )(h, aq, akh, avh, aqrt)


def _tc_post(h, numv, numrd, wvrp, wop):
    def body(h_ref, nv_ref, nrd_ref, wvr_ref, wo_ref, o_ref):
        nrd = nrd_ref[...]
        num = nv_ref[...] + jnp.dot(nrd, wvr_ref[...],
                                    preferred_element_type=_f32)
        den = nrd[:, NB:NB + 1] + 1e-9
        agg = num / den
        t = h_ref[...] + jnp.dot(agg, wo_ref[...],
                                 preferred_element_type=_f32)
        s1 = jnp.sum(t, axis=-1, keepdims=True)
        s2 = jnp.sum(t * t, axis=-1, keepdims=True)
        mu = s1 / D
        var = s2 / D - mu * mu
        lnv = (t - mu) / jnp.sqrt(var + 1e-5)
        cmask = (lax.broadcasted_iota(_i32, (1, DP), 1) < D).astype(_f32)
        o_ref[...] = lnv * cmask

    return pl.pallas_call(
        body,
        grid=(NPAD // RB,),
        in_specs=[
            pl.BlockSpec((RB, DP), lambda i: (i, 0)),
            pl.BlockSpec((RB, DP), lambda i: (i, 0)),
            pl.BlockSpec((RB, NRB), lambda i: (i, 0)),
            pl.BlockSpec((NRB, DP), lambda i: (0, 0)),
            pl.BlockSpec((DP, DP), lambda i: (0, 0)),
        ],
        out_specs=pl.BlockSpec((RB, DP), lambda i: (i, 0)),
        out_shape=jax.ShapeDtypeStruct((NPAD, DP), _f32),
    )(h, numv, numrd, wvrp, wop)


def _tc_final(h, wwp):
    def body(h_ref, ww_ref, o_ref):
        hb = h_ref[...]
        lg = jnp.dot(hb, ww_ref[...], preferred_element_type=_f32)
        gate = 1.0 / (1.0 + jnp.exp(-lg[:, 0:1]))
        o_ref[...] = hb * gate

    return pl.pallas_call(
        body,
        grid=(NPAD // RB,),
        in_specs=[
            pl.BlockSpec((RB, DP), lambda i: (i, 0)),
            pl.BlockSpec((DP, NRB), lambda i: (0, 0)),
        ],
        out_specs=pl.BlockSpec((RB, DP), lambda i: (i, 0)),
        out_shape=jax.ShapeDtypeStruct((NPAD, DP), _f32),
    )(h, wwp)


def kernel(x, pos, edge_index, W_emb, Wq, Wk, Wv, Wo, Ww):
    src = edge_index[0].astype(_i32)
    dst = edge_index[1].astype(_i32)
    src_in = jnp.pad(src, (0, EIN - E))
    dst_in = jnp.pad(dst, (0, EIN - E))
    posp = jnp.pad(pos, ((0, 0), (0, 16 - 3)))
    xp = jnp.pad(x, ((0, NPAD - N), (0, 8 - D_IN)))
    wembp = jnp.pad(W_emb, ((0, 8 - D_IN), (0, DP - D)))

    cnt = _k_hist(dst_in)
    srcb, dstb, offp, deg = _k_place(dst_in, src_in, cnt.reshape(-1))
    dist = _k_dist(srcb, dstb, posp)

    h = _tc_embed(xp, wembp)
    for l in range(L):
        aq = jnp.pad(Wq[l], ((0, DP - D), (0, DP - D)))
        akh = jnp.pad(Wk[l][:D], ((0, DP - D), (0, DP - D)))
        avh = jnp.pad(Wv[l][:D], ((0, DP - D), (0, DP - D)))
        aqrt = jnp.pad(Wk[l][D:].T, ((0, DP - D), (0, NRB - NB)))
        wvrp = jnp.pad(Wv[l][D:], ((0, NRB - NB), (0, DP - D)))
        wop = jnp.pad(Wo[l], ((0, DP - D), (0, DP - D)))

        q, qr, kn, vn = _tc_pre(h, aq, akh, avh, aqrt)
        numv, numrd = _k_sweep(q, qr, kn, vn, srcb, dstb, dist, offp, deg)
        h = _tc_post(h, numv, numrd, wvrp, wop)

    wwp = jnp.pad(Ww, ((0, DP - D), (0, NRB - 1)))
    out = _tc_final(h, wwp)
    return out[:N, :D]


# speedup vs baseline: 2.9998x; 1.1709x over previous
import functools

import jax
import jax.numpy as jnp
import numpy as np
from jax import lax
from jax.experimental import pallas as pl
from jax.experimental.pallas import tpu as pltpu
from jax.experimental.pallas import tpu_sc as plsc

N, E, D, D_IN, NB, L = 50000, 800000, 86, 3, 10, 3
MAX_RADIUS = 2.0
SCALE = 1.0 / np.sqrt(D)

DP = 96
NRB = 16
NPB = 256
NBKT = 196
BKT_PAD = 208
NPAD = NBKT * NPB
NWRK = 32
EWP = 25088
EIN = NWRK * EWP
EPAD = 802816
EDW = EPAD // NWRK
CH = 128
RB = 1792

_mesh = plsc.VectorSubcoreMesh(core_axis_name="c", subcore_axis_name="s")
_f32 = jnp.float32
_i32 = jnp.int32


def _wid():
    return lax.axis_index("s") * 2 + lax.axis_index("c")


def _iota16():
    return lax.broadcasted_iota(_i32, (16,), 0)


def _full(v, dtype=_i32):
    return jnp.full((16,), v, dtype)


def _elem(ref, i):
    g = plsc.load_gather(ref, [_full(i)])
    return jnp.sum(jnp.where(_iota16() == 0, g, jnp.zeros_like(g)))


@functools.partial(
    pl.kernel,
    compiler_params=pltpu.CompilerParams(needs_layout_passes=False, use_tc_tiling_on_sc=False),
    out_type=jax.ShapeDtypeStruct((NWRK, BKT_PAD * 16), _i32),
    mesh=_mesh,
    scratch_types=[
        pltpu.VMEM((EWP,), _i32),
        pltpu.VMEM((BKT_PAD * 16,), _i32),
    ],
)
def _k_hist(dst_hbm, cnt_hbm, dst_v, cnt_v):
    w = _wid()
    pltpu.sync_copy(dst_hbm.at[pl.ds(pl.multiple_of(w * EWP, 128), EWP)], dst_v)
    zeros = jnp.zeros((16,), _i32)
    it16 = _iota16()

    def zb(i, _):
        cnt_v[pl.ds(i * 16, 16)] = zeros
        return 0

    lax.fori_loop(0, BKT_PAD, zb, 0)

    ebase = w * EWP

    def body(g, _):
        d16 = dst_v[pl.ds(g * 16, 16)]
        msk = (ebase + g * 16 + it16) < E
        idx = (d16 >> 8) * 16 + it16
        c = plsc.load_gather(cnt_v, [idx], mask=msk)
        plsc.store_scatter(cnt_v, [idx], c + 1, mask=msk)
        return 0

    lax.fori_loop(0, EWP // 16, body, 0)
    pltpu.sync_copy(cnt_v, cnt_hbm.at[w])


@functools.partial(
    pl.kernel,
    compiler_params=pltpu.CompilerParams(needs_layout_passes=False, use_tc_tiling_on_sc=False),
    out_type=(
        jax.ShapeDtypeStruct((EPAD,), _i32),
        jax.ShapeDtypeStruct((EPAD,), _i32),
        jax.ShapeDtypeStruct((BKT_PAD,), _i32),
        jax.ShapeDtypeStruct((BKT_PAD,), _i32),
    ),
    mesh=_mesh,
    scratch_types=[
        pltpu.VMEM((NWRK * BKT_PAD * 16,), _i32),
        pltpu.VMEM((BKT_PAD * 16,), _i32),
        pltpu.VMEM((BKT_PAD,), _i32),
        pltpu.VMEM((BKT_PAD,), _i32),
        pltpu.VMEM((CH,), _i32),
        pltpu.VMEM((CH,), _i32),
        pltpu.VMEM((CH,), _i32),
        pltpu.SemaphoreType.DMA,
    ],
)
def _k_place(dst_hbm, src_hbm, cntf_hbm, srcp_hbm, dstp_hbm, offp_hbm, deg_hbm,
             cnt_a, base16, deg_v, offp_v, srcv, dstv, posv, sem):
    w = _wid()
    pltpu.sync_copy(cntf_hbm, cnt_a)
    it16 = _iota16()
    zeros = jnp.zeros((16,), _i32)

    def degb(b, _):
        def accw(wi, s):
            return s + cnt_a[pl.ds((wi * BKT_PAD + b) * 16, 16)]

        tot = lax.fori_loop(0, NWRK, accw, zeros)
        s = jnp.sum(tot)
        plsc.store_scatter(deg_v, [_full(b)], _full(s), mask=it16 == 0)
        return 0

    lax.fori_loop(0, BKT_PAD, degb, 0)

    def pfx(gi, carry):
        d16 = deg_v[pl.ds(gi * 16, 16)]
        r16 = (d16 + 7) & _full(-8)
        cs = plsc.cumsum(r16)
        offp_v[pl.ds(gi * 16, 16)] = carry + cs - r16
        return carry + jnp.sum(r16)

    lax.fori_loop(0, BKT_PAD // 16, pfx, jnp.int32(0))

    @pl.when(w == 0)
    def _():
        pltpu.sync_copy(offp_v, offp_hbm)
        pltpu.sync_copy(deg_v, deg_hbm)

    def baseb(b, _):
        def accw(wi, s):
            row = cnt_a[pl.ds((wi * BKT_PAD + b) * 16, 16)]
            return s + jnp.where(wi < w, jnp.sum(row), 0)

        before = lax.fori_loop(0, NWRK, accw, jnp.int32(0))
        myrow = cnt_a[pl.ds((w * BKT_PAD + b) * 16, 16)]
        mycs = plsc.cumsum(myrow) - myrow
        ob = plsc.load_gather(offp_v, [_full(b)])
        base16[pl.ds(b * 16, 16)] = ob + before + mycs
        return 0

    lax.fori_loop(0, BKT_PAD, baseb, 0)

    dump = _full(EPAD - 16) + it16

    def chunk(ci, _):
        cb = pl.multiple_of(w * EWP + ci * CH, 128)
        pltpu.sync_copy(dst_hbm.at[pl.ds(cb, CH)], dstv)
        pltpu.sync_copy(src_hbm.at[pl.ds(cb, CH)], srcv)

        def place(g, _):
            d16 = dstv[pl.ds(g * 16, 16)]
            msk = (cb + g * 16 + it16) < E
            idx = (d16 >> 8) * 16 + it16
            p = plsc.load_gather(base16, [idx], mask=msk)
            plsc.store_scatter(base16, [idx], p + 1, mask=msk)
            posv[pl.ds(g * 16, 16)] = jnp.where(msk, p, dump)
            return 0

        lax.fori_loop(0, CH // 16, place, 0)
        pltpu.async_copy(srcv, srcp_hbm.at[posv], sem).wait()
        pltpu.async_copy(dstv, dstp_hbm.at[posv], sem).wait()
        return 0

    lax.fori_loop(0, EWP // CH, chunk, 0)


@functools.partial(
    pl.kernel,
    compiler_params=pltpu.CompilerParams(needs_layout_passes=False, use_tc_tiling_on_sc=False),
    out_type=jax.ShapeDtypeStruct((EPAD,), _f32),
    mesh=_mesh,
    scratch_types=[
        pltpu.VMEM((CH,), _i32),
        pltpu.VMEM((CH,), _i32),
        pltpu.VMEM((CH, 16), _f32),
        pltpu.VMEM((CH, 16), _f32),
        pltpu.VMEM((CH,), _f32),
        pltpu.SemaphoreType.DMA,
        pltpu.SemaphoreType.DMA,
    ],
)
def _k_dist(srcp_hbm, dstp_hbm, pos_hbm, dist_hbm, sv, dv, ps, pd, dout,
            sem1, sem2):
    w = _wid()
    it16 = _iota16()
    nmax = _full(N - 1)
    zeroi = jnp.zeros((16,), _i32)
    magic = _full(0x1FBD1DF5)
    half = _full(0.5, _f32)
    eps = _full(1e-12, _f32)

    def chunk(ci, _):
        e0 = pl.multiple_of(w * EDW + ci * CH, 128)
        pltpu.sync_copy(srcp_hbm.at[pl.ds(e0, CH)], sv)
        pltpu.sync_copy(dstp_hbm.at[pl.ds(e0, CH)], dv)

        def clampg(g, _):
            sv[pl.ds(g * 16, 16)] = jnp.minimum(
                jnp.maximum(sv[pl.ds(g * 16, 16)], zeroi), nmax)
            dv[pl.ds(g * 16, 16)] = jnp.minimum(
                jnp.maximum(dv[pl.ds(g * 16, 16)], zeroi), nmax)
            return 0

        lax.fori_loop(0, CH // 16, clampg, 0)
        pltpu.async_copy(pos_hbm.at[sv], ps, sem1).wait()
        pltpu.async_copy(pos_hbm.at[dv], pd, sem2).wait()

        def dot3(g, _):
            rows = g * 16 + it16
            s = eps
            for cdim in range(3):
                a = plsc.load_gather(ps, [rows, _full(cdim)])
                b = plsc.load_gather(pd, [rows, _full(cdim)])
                d = a - b
                s = s + d * d
            i = plsc.bitcast(s, _i32)
            y = plsc.bitcast(magic + (i >> 1), _f32)
            for _ in range(3):
                y = half * (y + s / y)
            dout[pl.ds(g * 16, 16)] = y
            return 0

        lax.fori_loop(0, CH // 16, dot3, 0)
        pltpu.sync_copy(dout, dist_hbm.at[pl.ds(e0, CH)])
        return 0

    lax.fori_loop(0, EDW // CH, chunk, 0)


_RBF_C = np.linspace(0.0, MAX_RADIUS, NB)
_RBF_I = 1.0 / (2.0 * (MAX_RADIUS / NB) ** 2)


@functools.partial(
    pl.kernel,
    compiler_params=pltpu.CompilerParams(needs_layout_passes=False, use_tc_tiling_on_sc=False),
    out_type=(
        jax.ShapeDtypeStruct((NPAD, DP), _f32),
        jax.ShapeDtypeStruct((NPAD, NRB), _f32),
    ),
    mesh=_mesh,
    scratch_types=[
        pltpu.VMEM((NPB, DP), _f32),
        pltpu.VMEM((NPB, NRB), _f32),
        pltpu.VMEM((CH, DP), _f32),
        pltpu.VMEM((CH, DP), _f32),
        pltpu.VMEM((CH, NRB), _f32),
        pltpu.VMEM((CH,), _i32),
        pltpu.VMEM((CH,), _i32),
        pltpu.VMEM((CH,), _f32),
        pltpu.VMEM((CH,), _i32),
        pltpu.VMEM((CH,), _i32),
        pltpu.VMEM((CH,), _i32),
        pltpu.VMEM((BKT_PAD,), _i32),
        pltpu.VMEM((BKT_PAD,), _i32),
        pltpu.VMEM((CH, DP), _f32),
        pltpu.VMEM_SHARED((16 * NPB, DP), _f32),
        pltpu.VMEM_SHARED((16 * NPB, NRB), _f32),
        pltpu.SemaphoreType.DMA,
        pltpu.SemaphoreType.DMA,
        pltpu.SemaphoreType.DMA,
        pltpu.SemaphoreType.DMA,
    ],
)
def _k_sweep(q_hbm, qr_hbm, kt_hbm, vt_hbm, srcp_hbm, dstp_hbm, dist_hbm,
             offp_hbm, deg_hbm, numv_hbm, numrd_hbm,
             q_l, qr_l, kc, vc, urd, srcv, dstv, distv, sidx, aidx, dlv,
             offp_v, deg_v, zblk, accv, accrd, semk, semv, sema, semb):
    w = _wid()
    sid = lax.axis_index("s")
    it16 = _iota16()
    pltpu.sync_copy(offp_hbm, offp_v)
    pltpu.sync_copy(deg_hbm, deg_v)

    zeros = jnp.zeros((16,), _f32)
    zeroi = jnp.zeros((16,), _i32)
    nmax = _full(N - 1)
    npbm1 = _full(NPB - 1)

    def zrow(r, _):
        for c in range(DP // 16):
            zblk[r, pl.ds(c * 16, 16)] = zeros
        return 0

    lax.fori_loop(0, CH, zrow, 0)

    for t in range(7):
        b = w + NWRK * t

        @pl.when(b < NBKT)
        def _():
            pltpu.sync_copy(q_hbm.at[pl.ds(pl.multiple_of(b * NPB, NPB), NPB)], q_l)
            pltpu.sync_copy(qr_hbm.at[pl.ds(pl.multiple_of(b * NPB, NPB), NPB)], qr_l)
            for z in range(NPB // CH):
                pltpu.sync_copy(zblk, accv.at[pl.ds(sid * NPB + z * CH, CH)])
                pltpu.sync_copy(zblk.at[:, pl.ds(0, NRB)],
                                accrd.at[pl.ds(sid * NPB + z * CH, CH)])

            degb = _elem(deg_v, b)
            offb = _elem(offp_v, b)
            nch = (degb + CH - 1) // CH

            def chunk(ci, _):
                e0 = pl.multiple_of(offb + ci * CH, 8)
                pltpu.sync_copy(srcp_hbm.at[pl.ds(e0, CH)], srcv)
                pltpu.sync_copy(dstp_hbm.at[pl.ds(e0, CH)], dstv)
                pltpu.sync_copy(dist_hbm.at[pl.ds(e0, CH)], distv)

                def cleang(g, _):
                    s16 = srcv[pl.ds(g * 16, 16)]
                    d16 = dstv[pl.ds(g * 16, 16)]
                    sidx[pl.ds(g * 16, 16)] = jnp.minimum(
                        jnp.maximum(s16, zeroi), nmax)
                    dl = jnp.minimum(jnp.maximum(d16 - b * NPB, zeroi), npbm1)
                    dlv[pl.ds(g * 16, 16)] = dl
                    aidx[pl.ds(g * 16, 16)] = dl + sid * NPB
                    return 0

                lax.fori_loop(0, CH // 16, cleang, 0)
                cpk = pltpu.async_copy(kt_hbm.at[sidx], kc, semk)
                cpv = pltpu.async_copy(vt_hbm.at[sidx], vc, semv)
                cpk.wait()

                def logitg(g, _):
                    dlg = dlv[pl.ds(g * 16, 16)]
                    rows = g * 16 + it16

                    def dotd(dd, accs):
                        a0, a1 = accs
                        qv0 = plsc.load_gather(q_l, [dlg, _full(2 * dd)])
                        kv0 = plsc.load_gather(kc, [rows, _full(2 * dd)])
                        qv1 = plsc.load_gather(q_l, [dlg, _full(2 * dd + 1)])
                        kv1 = plsc.load_gather(kc, [rows, _full(2 * dd + 1)])
                        return (a0 + qv0 * kv0, a1 + qv1 * kv1)

                    acc0, acc1 = lax.fori_loop(0, D // 2, dotd,
                                               (zeros, zeros), unroll=8)
                    acc = acc0 + acc1

                    dg = distv[pl.ds(g * 16, 16)]
                    acc2 = zeros
                    rjs = []
                    for j in range(NB):
                        dd = dg - _full(_RBF_C[j], _f32)
                        rj = jnp.exp(dd * dd * _full(-_RBF_I, _f32))
                        qrv = plsc.load_gather(qr_l, [dlg, _full(j)])
                        acc2 = acc2 + qrv * rj
                        rjs.append(rj)
                    lg = acc + acc2
                    valid = (ci * CH + g * 16 + it16) < degb
                    ex = jnp.where(valid, jnp.exp(lg), zeros)
                    for j in range(NB):
                        plsc.store_scatter(urd, [rows, _full(j)], rjs[j] * ex)
                    plsc.store_scatter(urd, [rows, _full(NB)], ex)
                    for j in range(NB + 1, NRB):
                        plsc.store_scatter(urd, [rows, _full(j)], zeros)

                    def upd(dd, exc):
                        vv0 = plsc.load_gather(vc, [rows, _full(2 * dd)])
                        vv1 = plsc.load_gather(vc, [rows, _full(2 * dd + 1)])
                        plsc.store_scatter(vc, [rows, _full(2 * dd)],
                                           vv0 * exc)
                        plsc.store_scatter(vc, [rows, _full(2 * dd + 1)],
                                           vv1 * exc)
                        return exc

                    lax.fori_loop(0, D // 2, upd, ex, unroll=8)
                    return 0

                cpv.wait()
                lax.fori_loop(0, CH // 16, logitg, 0)
                pltpu.async_copy(vc, accv.at[aidx], sema, add=True).wait()
                pltpu.async_copy(urd, accrd.at[aidx], semb, add=True).wait()
                return 0

            lax.fori_loop(0, nch, chunk, 0)
            pltpu.sync_copy(accv.at[pl.ds(sid * NPB, NPB)],
                            numv_hbm.at[pl.ds(pl.multiple_of(b * NPB, NPB), NPB)])
            pltpu.sync_copy(accrd.at[pl.ds(sid * NPB, NPB)],
                            numrd_hbm.at[pl.ds(pl.multiple_of(b * NPB, NPB), NPB)])


def _tc_embed(xp, wembp):
    def body(x_ref, w_ref, o_ref):
        o_ref[...] = jnp.dot(x_ref[...], w_ref[...],
                             preferred_element_type=_f32)

    return pl.pallas_call(
        body,
        grid=(NPAD // RB,),
        in_specs=[
            pl.BlockSpec((RB, 8), lambda i: (i, 0)),
            pl.BlockSpec((8, DP), lambda i: (0, 0)),
        ],
        out_specs=pl.BlockSpec((RB, DP), lambda i: (i, 0)),
        out_shape=jax.ShapeDtypeStruct((NPAD, DP), _f32),
    )(xp, wembp)


def _tc_pre(h, aq, akh, avh, aqrt):
    def body(h_ref, aq_ref, akh_ref, avh_ref, aqrt_ref,
             q_ref, qr_ref, kn_ref, vn_ref):
        hb = h_ref[...]
        q = jnp.dot(hb, aq_ref[...], preferred_element_type=_f32) * SCALE
        q_ref[...] = q
        qr_ref[...] = jnp.dot(q, aqrt_ref[...], preferred_element_type=_f32)
        kn_ref[...] = jnp.dot(hb, akh_ref[...], preferred_element_type=_f32)
        vn_ref[...] = jnp.dot(hb, avh_ref[...], preferred_element_type=_f32)

    return pl.pallas_call(
        body,
        grid=(NPAD // RB,),
        in_specs=[
            pl.BlockSpec((RB, DP), lambda i: (i, 0)),
            pl.BlockSpec((DP, DP), lambda i: (0, 0)),
            pl.BlockSpec((DP, DP), lambda i: (0, 0)),
            pl.BlockSpec((DP, DP), lambda i: (0, 0)),
            pl.BlockSpec((DP, NRB), lambda i: (0, 0)),
        ],
        out_specs=[
            pl.BlockSpec((RB, DP), lambda i: (i, 0)),
            pl.BlockSpec((RB, NRB), lambda i: (i, 0)),
            pl.BlockSpec((RB, DP), lambda i: (i, 0)),
            pl.BlockSpec((RB, DP), lambda i: (i, 0)),
        ],
        out_shape=(
            jax.ShapeDtypeStruct((NPAD, DP), _f32),
            jax.ShapeDtypeStruct((NPAD, NRB), _f32),
            jax.ShapeDtypeStruct((NPAD, DP), _f32),
            jax.ShapeDtypeStruct((NPAD, DP), _f32),
        ),
    )(h, aq, akh, avh, aqrt)


def _tc_post(h, numv, numrd, wvrp, wop):
    def body(h_ref, nv_ref, nrd_ref, wvr_ref, wo_ref, o_ref):
        nrd = nrd_ref[...]
        num = nv_ref[...] + jnp.dot(nrd, wvr_ref[...],
                                    preferred_element_type=_f32)
        den = nrd[:, NB:NB + 1] + 1e-9
        agg = num / den
        t = h_ref[...] + jnp.dot(agg, wo_ref[...],
                                 preferred_element_type=_f32)
        s1 = jnp.sum(t, axis=-1, keepdims=True)
        s2 = jnp.sum(t * t, axis=-1, keepdims=True)
        mu = s1 / D
        var = s2 / D - mu * mu
        lnv = (t - mu) / jnp.sqrt(var + 1e-5)
        cmask = (lax.broadcasted_iota(_i32, (1, DP), 1) < D).astype(_f32)
        o_ref[...] = lnv * cmask

    return pl.pallas_call(
        body,
        grid=(NPAD // RB,),
        in_specs=[
            pl.BlockSpec((RB, DP), lambda i: (i, 0)),
            pl.BlockSpec((RB, DP), lambda i: (i, 0)),
            pl.BlockSpec((RB, NRB), lambda i: (i, 0)),
            pl.BlockSpec((NRB, DP), lambda i: (0, 0)),
            pl.BlockSpec((DP, DP), lambda i: (0, 0)),
        ],
        out_specs=pl.BlockSpec((RB, DP), lambda i: (i, 0)),
        out_shape=jax.ShapeDtypeStruct((NPAD, DP), _f32),
    )(h, numv, numrd, wvrp, wop)


def _tc_final(h, wwp):
    def body(h_ref, ww_ref, o_ref):
        hb = h_ref[...]
        lg = jnp.dot(hb, ww_ref[...], preferred_element_type=_f32)
        gate = 1.0 / (1.0 + jnp.exp(-lg[:, 0:1]))
        o_ref[...] = hb * gate

    return pl.pallas_call(
        body,
        grid=(NPAD // RB,),
        in_specs=[
            pl.BlockSpec((RB, DP), lambda i: (i, 0)),
            pl.BlockSpec((DP, NRB), lambda i: (0, 0)),
        ],
        out_specs=pl.BlockSpec((RB, DP), lambda i: (i, 0)),
        out_shape=jax.ShapeDtypeStruct((NPAD, DP), _f32),
    )(h, wwp)


def kernel(x, pos, edge_index, W_emb, Wq, Wk, Wv, Wo, Ww):
    src = edge_index[0].astype(_i32)
    dst = edge_index[1].astype(_i32)
    src_in = jnp.pad(src, (0, EIN - E))
    dst_in = jnp.pad(dst, (0, EIN - E))
    posp = jnp.pad(pos, ((0, 0), (0, 16 - 3)))
    xp = jnp.pad(x, ((0, NPAD - N), (0, 8 - D_IN)))
    wembp = jnp.pad(W_emb, ((0, 8 - D_IN), (0, DP - D)))

    cnt = _k_hist(dst_in)
    srcb, dstb, offp, deg = _k_place(dst_in, src_in, cnt.reshape(-1))
    dist = _k_dist(srcb, dstb, posp)

    h = _tc_embed(xp, wembp)
    for l in range(L):
        aq = jnp.pad(Wq[l], ((0, DP - D), (0, DP - D)))
        akh = jnp.pad(Wk[l][:D], ((0, DP - D), (0, DP - D)))
        avh = jnp.pad(Wv[l][:D], ((0, DP - D), (0, DP - D)))
        aqrt = jnp.pad(Wk[l][D:].T, ((0, DP - D), (0, NRB - NB)))
        wvrp = jnp.pad(Wv[l][D:], ((0, NRB - NB), (0, DP - D)))
        wop = jnp.pad(Wo[l], ((0, DP - D), (0, DP - D)))

        q, qr, kn, vn = _tc_pre(h, aq, akh, avh, aqrt)
        numv, numrd = _k_sweep(q, qr, kn, vn, srcb, dstb, dist, offp, deg)
        h = _tc_post(h, numv, numrd, wvrp, wop)

    wwp = jnp.pad(Ww, ((0, DP - D), (0, NRB - 1)))
    out = _tc_final(h, wwp)
    return out[:N, :D]


# speedup vs baseline: 3.1461x; 1.0487x over previous
import functools

import jax
import jax.numpy as jnp
import numpy as np
from jax import lax
from jax.experimental import pallas as pl
from jax.experimental.pallas import tpu as pltpu
from jax.experimental.pallas import tpu_sc as plsc

N, E, D, D_IN, NB, L = 50000, 800000, 86, 3, 10, 3
MAX_RADIUS = 2.0
SCALE = 1.0 / np.sqrt(D)

DP = 96
NRB = 16
NPB = 256
NBKT = 196
BKT_PAD = 208
NPAD = NBKT * NPB
NWRK = 32
EWP = 25088
EIN = NWRK * EWP
EPAD = 802816
EDW = EPAD // NWRK
CH = 128
RB = 1792

_mesh = plsc.VectorSubcoreMesh(core_axis_name="c", subcore_axis_name="s")
_f32 = jnp.float32
_i32 = jnp.int32


def _wid():
    return lax.axis_index("s") * 2 + lax.axis_index("c")


def _iota16():
    return lax.broadcasted_iota(_i32, (16,), 0)


def _full(v, dtype=_i32):
    return jnp.full((16,), v, dtype)


def _elem(ref, i):
    g = plsc.load_gather(ref, [_full(i)])
    return jnp.sum(jnp.where(_iota16() == 0, g, jnp.zeros_like(g)))


@functools.partial(
    pl.kernel,
    compiler_params=pltpu.CompilerParams(needs_layout_passes=False, use_tc_tiling_on_sc=False),
    out_type=jax.ShapeDtypeStruct((NWRK, BKT_PAD * 16), _i32),
    mesh=_mesh,
    scratch_types=[
        pltpu.VMEM((EWP,), _i32),
        pltpu.VMEM((BKT_PAD * 16,), _i32),
    ],
)
def _k_hist(dst_hbm, cnt_hbm, dst_v, cnt_v):
    w = _wid()
    pltpu.sync_copy(dst_hbm.at[pl.ds(pl.multiple_of(w * EWP, 128), EWP)], dst_v)
    zeros = jnp.zeros((16,), _i32)
    it16 = _iota16()

    def zb(i, _):
        cnt_v[pl.ds(i * 16, 16)] = zeros
        return 0

    lax.fori_loop(0, BKT_PAD, zb, 0)

    ebase = w * EWP

    def body(g, _):
        d16 = dst_v[pl.ds(g * 16, 16)]
        msk = (ebase + g * 16 + it16) < E
        idx = (d16 >> 8) * 16 + it16
        c = plsc.load_gather(cnt_v, [idx], mask=msk)
        plsc.store_scatter(cnt_v, [idx], c + 1, mask=msk)
        return 0

    lax.fori_loop(0, EWP // 16, body, 0)
    pltpu.sync_copy(cnt_v, cnt_hbm.at[w])


@functools.partial(
    pl.kernel,
    compiler_params=pltpu.CompilerParams(needs_layout_passes=False, use_tc_tiling_on_sc=False),
    out_type=(
        jax.ShapeDtypeStruct((EPAD,), _i32),
        jax.ShapeDtypeStruct((EPAD,), _i32),
        jax.ShapeDtypeStruct((BKT_PAD,), _i32),
        jax.ShapeDtypeStruct((BKT_PAD,), _i32),
    ),
    mesh=_mesh,
    scratch_types=[
        pltpu.VMEM((NWRK * BKT_PAD * 16,), _i32),
        pltpu.VMEM((BKT_PAD * 16,), _i32),
        pltpu.VMEM((BKT_PAD,), _i32),
        pltpu.VMEM((BKT_PAD,), _i32),
        pltpu.VMEM((CH,), _i32),
        pltpu.VMEM((CH,), _i32),
        pltpu.VMEM((CH,), _i32),
        pltpu.SemaphoreType.DMA,
    ],
)
def _k_place(dst_hbm, src_hbm, cntf_hbm, srcp_hbm, dstp_hbm, offp_hbm, deg_hbm,
             cnt_a, base16, deg_v, offp_v, srcv, dstv, posv, sem):
    w = _wid()
    pltpu.sync_copy(cntf_hbm, cnt_a)
    it16 = _iota16()
    zeros = jnp.zeros((16,), _i32)

    def degb(b, _):
        def accw(wi, s):
            return s + cnt_a[pl.ds((wi * BKT_PAD + b) * 16, 16)]

        tot = lax.fori_loop(0, NWRK, accw, zeros)
        s = jnp.sum(tot)
        plsc.store_scatter(deg_v, [_full(b)], _full(s), mask=it16 == 0)
        return 0

    lax.fori_loop(0, BKT_PAD, degb, 0)

    def pfx(gi, carry):
        d16 = deg_v[pl.ds(gi * 16, 16)]
        r16 = (d16 + 7) & _full(-8)
        cs = plsc.cumsum(r16)
        offp_v[pl.ds(gi * 16, 16)] = carry + cs - r16
        return carry + jnp.sum(r16)

    lax.fori_loop(0, BKT_PAD // 16, pfx, jnp.int32(0))

    @pl.when(w == 0)
    def _():
        pltpu.sync_copy(offp_v, offp_hbm)
        pltpu.sync_copy(deg_v, deg_hbm)

    def baseb(b, _):
        def accw(wi, s):
            row = cnt_a[pl.ds((wi * BKT_PAD + b) * 16, 16)]
            return s + jnp.where(wi < w, jnp.sum(row), 0)

        before = lax.fori_loop(0, NWRK, accw, jnp.int32(0))
        myrow = cnt_a[pl.ds((w * BKT_PAD + b) * 16, 16)]
        mycs = plsc.cumsum(myrow) - myrow
        ob = plsc.load_gather(offp_v, [_full(b)])
        base16[pl.ds(b * 16, 16)] = ob + before + mycs
        return 0

    lax.fori_loop(0, BKT_PAD, baseb, 0)

    dump = _full(EPAD - 16) + it16

    def chunk(ci, _):
        cb = pl.multiple_of(w * EWP + ci * CH, 128)
        pltpu.sync_copy(dst_hbm.at[pl.ds(cb, CH)], dstv)
        pltpu.sync_copy(src_hbm.at[pl.ds(cb, CH)], srcv)

        def place(g, _):
            d16 = dstv[pl.ds(g * 16, 16)]
            msk = (cb + g * 16 + it16) < E
            idx = (d16 >> 8) * 16 + it16
            p = plsc.load_gather(base16, [idx], mask=msk)
            plsc.store_scatter(base16, [idx], p + 1, mask=msk)
            posv[pl.ds(g * 16, 16)] = jnp.where(msk, p, dump)
            return 0

        lax.fori_loop(0, CH // 16, place, 0)
        pltpu.async_copy(srcv, srcp_hbm.at[posv], sem).wait()
        pltpu.async_copy(dstv, dstp_hbm.at[posv], sem).wait()
        return 0

    lax.fori_loop(0, EWP // CH, chunk, 0)


@functools.partial(
    pl.kernel,
    compiler_params=pltpu.CompilerParams(needs_layout_passes=False, use_tc_tiling_on_sc=False),
    out_type=jax.ShapeDtypeStruct((EPAD,), _f32),
    mesh=_mesh,
    scratch_types=[
        pltpu.VMEM((CH,), _i32),
        pltpu.VMEM((CH,), _i32),
        pltpu.VMEM((CH, 16), _f32),
        pltpu.VMEM((CH, 16), _f32),
        pltpu.VMEM((CH,), _f32),
        pltpu.SemaphoreType.DMA,
        pltpu.SemaphoreType.DMA,
    ],
)
def _k_dist(srcp_hbm, dstp_hbm, pos_hbm, dist_hbm, sv, dv, ps, pd, dout,
            sem1, sem2):
    w = _wid()
    it16 = _iota16()
    nmax = _full(N - 1)
    zeroi = jnp.zeros((16,), _i32)
    magic = _full(0x1FBD1DF5)
    half = _full(0.5, _f32)
    eps = _full(1e-12, _f32)

    def chunk(ci, _):
        e0 = pl.multiple_of(w * EDW + ci * CH, 128)
        pltpu.sync_copy(srcp_hbm.at[pl.ds(e0, CH)], sv)
        pltpu.sync_copy(dstp_hbm.at[pl.ds(e0, CH)], dv)

        def clampg(g, _):
            sv[pl.ds(g * 16, 16)] = jnp.minimum(
                jnp.maximum(sv[pl.ds(g * 16, 16)], zeroi), nmax)
            dv[pl.ds(g * 16, 16)] = jnp.minimum(
                jnp.maximum(dv[pl.ds(g * 16, 16)], zeroi), nmax)
            return 0

        lax.fori_loop(0, CH // 16, clampg, 0)
        pltpu.async_copy(pos_hbm.at[sv], ps, sem1).wait()
        pltpu.async_copy(pos_hbm.at[dv], pd, sem2).wait()

        def dot3(g, _):
            rows = g * 16 + it16
            s = eps
            for cdim in range(3):
                a = plsc.load_gather(ps, [rows, _full(cdim)])
                b = plsc.load_gather(pd, [rows, _full(cdim)])
                d = a - b
                s = s + d * d
            i = plsc.bitcast(s, _i32)
            y = plsc.bitcast(magic + (i >> 1), _f32)
            for _ in range(3):
                y = half * (y + s / y)
            dout[pl.ds(g * 16, 16)] = y
            return 0

        lax.fori_loop(0, CH // 16, dot3, 0)
        pltpu.sync_copy(dout, dist_hbm.at[pl.ds(e0, CH)])
        return 0

    lax.fori_loop(0, EDW // CH, chunk, 0)


_RBF_C = np.linspace(0.0, MAX_RADIUS, NB)
_RBF_I = 1.0 / (2.0 * (MAX_RADIUS / NB) ** 2)


@functools.partial(
    pl.kernel,
    compiler_params=pltpu.CompilerParams(needs_layout_passes=False, use_tc_tiling_on_sc=False),
    out_type=(
        jax.ShapeDtypeStruct((NPAD, DP), _f32),
        jax.ShapeDtypeStruct((NPAD, NRB), _f32),
    ),
    mesh=_mesh,
    scratch_types=[
        pltpu.VMEM((NPB, DP), _f32),
        pltpu.VMEM((NPB, NRB), _f32),
        pltpu.VMEM((2, CH, DP), _f32),
        pltpu.VMEM((2, CH, DP), _f32),
        pltpu.VMEM((CH, NRB), _f32),
        pltpu.VMEM((2, CH), _i32),
        pltpu.VMEM((2, CH), _i32),
        pltpu.VMEM((2, CH), _f32),
        pltpu.VMEM((2, CH), _i32),
        pltpu.VMEM((2, CH), _i32),
        pltpu.VMEM((2, CH), _i32),
        pltpu.VMEM((BKT_PAD,), _i32),
        pltpu.VMEM((BKT_PAD,), _i32),
        pltpu.VMEM((CH, DP), _f32),
        pltpu.VMEM_SHARED((16 * NPB, DP), _f32),
        pltpu.VMEM_SHARED((16 * NPB, NRB), _f32),
        pltpu.SemaphoreType.DMA,
        pltpu.SemaphoreType.DMA,
        pltpu.SemaphoreType.DMA,
        pltpu.SemaphoreType.DMA,
    ],
)
def _k_sweep(q_hbm, qr_hbm, kt_hbm, vt_hbm, srcp_hbm, dstp_hbm, dist_hbm,
             offp_hbm, deg_hbm, numv_hbm, numrd_hbm,
             q_l, qr_l, kc, vc, urd, srcv, dstv, distv, sidx, aidx, dlv,
             offp_v, deg_v, zblk, accv, accrd, semk, semv, sema, semb):
    w = _wid()
    sid = lax.axis_index("s")
    it16 = _iota16()
    pltpu.sync_copy(offp_hbm, offp_v)
    pltpu.sync_copy(deg_hbm, deg_v)

    zeros = jnp.zeros((16,), _f32)
    zeroi = jnp.zeros((16,), _i32)
    nmax = _full(N - 1)
    npbm1 = _full(NPB - 1)

    def zrow(r, _):
        for c in range(DP // 16):
            zblk[r, pl.ds(c * 16, 16)] = zeros
        return 0

    lax.fori_loop(0, CH, zrow, 0)

    for t in range(7):
        b = w + NWRK * t

        @pl.when(b < NBKT)
        def _():
            pltpu.sync_copy(q_hbm.at[pl.ds(pl.multiple_of(b * NPB, NPB), NPB)], q_l)
            pltpu.sync_copy(qr_hbm.at[pl.ds(pl.multiple_of(b * NPB, NPB), NPB)], qr_l)
            for z in range(NPB // CH):
                pltpu.sync_copy(zblk, accv.at[pl.ds(sid * NPB + z * CH, CH)])
                pltpu.sync_copy(zblk.at[:, pl.ds(0, NRB)],
                                accrd.at[pl.ds(sid * NPB + z * CH, CH)])

            degb = _elem(deg_v, b)
            offb = _elem(offp_v, b)
            nch = (degb + CH - 1) // CH

            def stage(ci, p):
                e0 = pl.multiple_of(offb + ci * CH, 8)
                pltpu.sync_copy(srcp_hbm.at[pl.ds(e0, CH)], srcv.at[p])
                pltpu.sync_copy(dstp_hbm.at[pl.ds(e0, CH)], dstv.at[p])
                pltpu.sync_copy(dist_hbm.at[pl.ds(e0, CH)], distv.at[p])

                def cleang(g, _):
                    s16 = srcv[p, pl.ds(g * 16, 16)]
                    d16 = dstv[p, pl.ds(g * 16, 16)]
                    sidx[p, pl.ds(g * 16, 16)] = jnp.minimum(
                        jnp.maximum(s16, zeroi), nmax)
                    dl = jnp.minimum(jnp.maximum(d16 - b * NPB, zeroi), npbm1)
                    dlv[p, pl.ds(g * 16, 16)] = dl
                    aidx[p, pl.ds(g * 16, 16)] = dl + sid * NPB
                    return 0

                lax.fori_loop(0, CH // 16, cleang, 0)
                pltpu.async_copy(kt_hbm.at[sidx.at[p]], kc.at[p], semk)
                pltpu.async_copy(vt_hbm.at[sidx.at[p]], vc.at[p], semv)

            @pl.when(nch > 0)
            def _():
                stage(jnp.int32(0), jnp.int32(0))

            def chunk(ci, _):
                p = lax.rem(ci, 2)

                @pl.when(ci + 1 < nch)
                def _():
                    stage(ci + 1, 1 - p)

                pltpu.make_async_copy(kt_hbm.at[pl.ds(0, CH)],
                                      kc.at[0], semk).wait()
                pltpu.make_async_copy(vt_hbm.at[pl.ds(0, CH)],
                                      vc.at[0], semv).wait()

                def logitg(g, _):
                    dlg = dlv[p, pl.ds(g * 16, 16)]
                    rows = g * 16 + it16
                    pp = _full(p)

                    def dotd(dd, accs):
                        a0, a1 = accs
                        qv0 = plsc.load_gather(q_l, [dlg, _full(2 * dd)])
                        kv0 = plsc.load_gather(kc, [pp, rows, _full(2 * dd)])
                        qv1 = plsc.load_gather(q_l, [dlg, _full(2 * dd + 1)])
                        kv1 = plsc.load_gather(kc, [pp, rows, _full(2 * dd + 1)])
                        return (a0 + qv0 * kv0, a1 + qv1 * kv1)

                    acc0, acc1 = lax.fori_loop(0, D // 2, dotd,
                                               (zeros, zeros), unroll=8)
                    acc = acc0 + acc1

                    dg = distv[p, pl.ds(g * 16, 16)]
                    acc2 = zeros
                    rjs = []
                    for j in range(NB):
                        dd = dg - _full(_RBF_C[j], _f32)
                        rj = jnp.exp(dd * dd * _full(-_RBF_I, _f32))
                        qrv = plsc.load_gather(qr_l, [dlg, _full(j)])
                        acc2 = acc2 + qrv * rj
                        rjs.append(rj)
                    lg = acc + acc2
                    valid = (ci * CH + g * 16 + it16) < degb
                    ex = jnp.where(valid, jnp.exp(lg), zeros)
                    for j in range(NB):
                        plsc.store_scatter(urd, [rows, _full(j)], rjs[j] * ex)
                    plsc.store_scatter(urd, [rows, _full(NB)], ex)
                    for j in range(NB + 1, NRB):
                        plsc.store_scatter(urd, [rows, _full(j)], zeros)

                    def upd(dd, exc):
                        vv0 = plsc.load_gather(vc, [pp, rows, _full(2 * dd)])
                        vv1 = plsc.load_gather(vc, [pp, rows, _full(2 * dd + 1)])
                        plsc.store_scatter(vc, [pp, rows, _full(2 * dd)],
                                           vv0 * exc)
                        plsc.store_scatter(vc, [pp, rows, _full(2 * dd + 1)],
                                           vv1 * exc)
                        return exc

                    lax.fori_loop(0, D // 2, upd, ex, unroll=8)
                    return 0

                lax.fori_loop(0, CH // 16, logitg, 0)
                pltpu.async_copy(vc.at[p], accv.at[aidx.at[p]], sema,
                                 add=True).wait()
                pltpu.async_copy(urd, accrd.at[aidx.at[p]], semb,
                                 add=True).wait()
                return 0

            lax.fori_loop(0, nch, chunk, 0)
            pltpu.sync_copy(accv.at[pl.ds(sid * NPB, NPB)],
                            numv_hbm.at[pl.ds(pl.multiple_of(b * NPB, NPB), NPB)])
            pltpu.sync_copy(accrd.at[pl.ds(sid * NPB, NPB)],
                            numrd_hbm.at[pl.ds(pl.multiple_of(b * NPB, NPB), NPB)])


def _tc_embed(xp, wembp):
    def body(x_ref, w_ref, o_ref):
        o_ref[...] = jnp.dot(x_ref[...], w_ref[...],
                             preferred_element_type=_f32)

    return pl.pallas_call(
        body,
        grid=(NPAD // RB,),
        in_specs=[
            pl.BlockSpec((RB, 8), lambda i: (i, 0)),
            pl.BlockSpec((8, DP), lambda i: (0, 0)),
        ],
        out_specs=pl.BlockSpec((RB, DP), lambda i: (i, 0)),
        out_shape=jax.ShapeDtypeStruct((NPAD, DP), _f32),
    )(xp, wembp)


def _tc_pre(h, aq, akh, avh, aqrt):
    def body(h_ref, aq_ref, akh_ref, avh_ref, aqrt_ref,
             q_ref, qr_ref, kn_ref, vn_ref):
        hb = h_ref[...]
        q = jnp.dot(hb, aq_ref[...], preferred_element_type=_f32) * SCALE
        q_ref[...] = q
        qr_ref[...] = jnp.dot(q, aqrt_ref[...], preferred_element_type=_f32)
        kn_ref[...] = jnp.dot(hb, akh_ref[...], preferred_element_type=_f32)
        vn_ref[...] = jnp.dot(hb, avh_ref[...], preferred_element_type=_f32)

    return pl.pallas_call(
        body,
        grid=(NPAD // RB,),
        in_specs=[
            pl.BlockSpec((RB, DP), lambda i: (i, 0)),
            pl.BlockSpec((DP, DP), lambda i: (0, 0)),
            pl.BlockSpec((DP, DP), lambda i: (0, 0)),
            pl.BlockSpec((DP, DP), lambda i: (0, 0)),
            pl.BlockSpec((DP, NRB), lambda i: (0, 0)),
        ],
        out_specs=[
            pl.BlockSpec((RB, DP), lambda i: (i, 0)),
            pl.BlockSpec((RB, NRB), lambda i: (i, 0)),
            pl.BlockSpec((RB, DP), lambda i: (i, 0)),
            pl.BlockSpec((RB, DP), lambda i: (i, 0)),
        ],
        out_shape=(
            jax.ShapeDtypeStruct((NPAD, DP), _f32),
            jax.ShapeDtypeStruct((NPAD, NRB), _f32),
            jax.ShapeDtypeStruct((NPAD, DP), _f32),
            jax.ShapeDtypeStruct((NPAD, DP), _f32),
        ),
    )(h, aq, akh, avh, aqrt)


def _tc_post(h, numv, numrd, wvrp, wop):
    def body(h_ref, nv_ref, nrd_ref, wvr_ref, wo_ref, o_ref):
        nrd = nrd_ref[...]
        num = nv_ref[...] + jnp.dot(nrd, wvr_ref[...],
                                    preferred_element_type=_f32)
        den = nrd[:, NB:NB + 1] + 1e-9
        agg = num / den
        t = h_ref[...] + jnp.dot(agg, wo_ref[...],
                                 preferred_element_type=_f32)
        s1 = jnp.sum(t, axis=-1, keepdims=True)
        s2 = jnp.sum(t * t, axis=-1, keepdims=True)
        mu = s1 / D
        var = s2 / D - mu * mu
        lnv = (t - mu) / jnp.sqrt(var + 1e-5)
        cmask = (lax.broadcasted_iota(_i32, (1, DP), 1) < D).astype(_f32)
        o_ref[...] = lnv * cmask

    return pl.pallas_call(
        body,
        grid=(NPAD // RB,),
        in_specs=[
            pl.BlockSpec((RB, DP), lambda i: (i, 0)),
            pl.BlockSpec((RB, DP), lambda i: (i, 0)),
            pl.BlockSpec((RB, NRB), lambda i: (i, 0)),
            pl.BlockSpec((NRB, DP), lambda i: (0, 0)),
            pl.BlockSpec((DP, DP), lambda i: (0, 0)),
        ],
        out_specs=pl.BlockSpec((RB, DP), lambda i: (i, 0)),
        out_shape=jax.ShapeDtypeStruct((NPAD, DP), _f32),
    )(h, numv, numrd, wvrp, wop)


def _tc_final(h, wwp):
    def body(h_ref, ww_ref, o_ref):
        hb = h_ref[...]
        lg = jnp.dot(hb, ww_ref[...], preferred_element_type=_f32)
        gate = 1.0 / (1.0 + jnp.exp(-lg[:, 0:1]))
        o_ref[...] = hb * gate

    return pl.pallas_call(
        body,
        grid=(NPAD // RB,),
        in_specs=[
            pl.BlockSpec((RB, DP), lambda i: (i, 0)),
            pl.BlockSpec((DP, NRB), lambda i: (0, 0)),
        ],
        out_specs=pl.BlockSpec((RB, DP), lambda i: (i, 0)),
        out_shape=jax.ShapeDtypeStruct((NPAD, DP), _f32),
    )(h, wwp)


def kernel(x, pos, edge_index, W_emb, Wq, Wk, Wv, Wo, Ww):
    src = edge_index[0].astype(_i32)
    dst = edge_index[1].astype(_i32)
    src_in = jnp.pad(src, (0, EIN - E))
    dst_in = jnp.pad(dst, (0, EIN - E))
    posp = jnp.pad(pos, ((0, 0), (0, 16 - 3)))
    xp = jnp.pad(x, ((0, NPAD - N), (0, 8 - D_IN)))
    wembp = jnp.pad(W_emb, ((0, 8 - D_IN), (0, DP - D)))

    cnt = _k_hist(dst_in)
    srcb, dstb, offp, deg = _k_place(dst_in, src_in, cnt.reshape(-1))
    dist = _k_dist(srcb, dstb, posp)

    h = _tc_embed(xp, wembp)
    for l in range(L):
        aq = jnp.pad(Wq[l], ((0, DP - D), (0, DP - D)))
        akh = jnp.pad(Wk[l][:D], ((0, DP - D), (0, DP - D)))
        avh = jnp.pad(Wv[l][:D], ((0, DP - D), (0, DP - D)))
        aqrt = jnp.pad(Wk[l][D:].T, ((0, DP - D), (0, NRB - NB)))
        wvrp = jnp.pad(Wv[l][D:], ((0, NRB - NB), (0, DP - D)))
        wop = jnp.pad(Wo[l], ((0, DP - D), (0, DP - D)))

        q, qr, kn, vn = _tc_pre(h, aq, akh, avh, aqrt)
        numv, numrd = _k_sweep(q, qr, kn, vn, srcb, dstb, dist, offp, deg)
        h = _tc_post(h, numv, numrd, wvrp, wop)

    wwp = jnp.pad(Ww, ((0, DP - D), (0, NRB - 1)))
    out = _tc_final(h, wwp)
    return out[:N, :D]


# speedup vs baseline: 3.1530x; 1.0022x over previous
import functools

import jax
import jax.numpy as jnp
import numpy as np
from jax import lax
from jax.experimental import pallas as pl
from jax.experimental.pallas import tpu as pltpu
from jax.experimental.pallas import tpu_sc as plsc

N, E, D, D_IN, NB, L = 50000, 800000, 86, 3, 10, 3
MAX_RADIUS = 2.0
SCALE = 1.0 / np.sqrt(D)

DP = 96
NRB = 16
NPB = 256
NBKT = 196
BKT_PAD = 208
NPAD = NBKT * NPB
NWRK = 32
EWP = 25088
EIN = NWRK * EWP
EPAD = 802816
EDW = EPAD // NWRK
CH = 128
RB = 1792

_mesh = plsc.VectorSubcoreMesh(core_axis_name="c", subcore_axis_name="s")
_f32 = jnp.float32
_i32 = jnp.int32


def _wid():
    return lax.axis_index("s") * 2 + lax.axis_index("c")


def _iota16():
    return lax.broadcasted_iota(_i32, (16,), 0)


def _full(v, dtype=_i32):
    return jnp.full((16,), v, dtype)


def _elem(ref, i):
    g = plsc.load_gather(ref, [_full(i)])
    return jnp.sum(jnp.where(_iota16() == 0, g, jnp.zeros_like(g)))


@functools.partial(
    pl.kernel,
    compiler_params=pltpu.CompilerParams(needs_layout_passes=False, use_tc_tiling_on_sc=False),
    out_type=jax.ShapeDtypeStruct((NWRK, BKT_PAD * 16), _i32),
    mesh=_mesh,
    scratch_types=[
        pltpu.VMEM((EWP,), _i32),
        pltpu.VMEM((BKT_PAD * 16,), _i32),
    ],
)
def _k_hist(dst_hbm, cnt_hbm, dst_v, cnt_v):
    w = _wid()
    pltpu.sync_copy(dst_hbm.at[pl.ds(pl.multiple_of(w * EWP, 128), EWP)], dst_v)
    zeros = jnp.zeros((16,), _i32)
    it16 = _iota16()

    def zb(i, _):
        cnt_v[pl.ds(i * 16, 16)] = zeros
        return 0

    lax.fori_loop(0, BKT_PAD, zb, 0)

    ebase = w * EWP

    def body(g, _):
        d16 = dst_v[pl.ds(g * 16, 16)]
        msk = (ebase + g * 16 + it16) < E
        idx = (d16 >> 8) * 16 + it16
        c = plsc.load_gather(cnt_v, [idx], mask=msk)
        plsc.store_scatter(cnt_v, [idx], c + 1, mask=msk)
        return 0

    lax.fori_loop(0, EWP // 16, body, 0)
    pltpu.sync_copy(cnt_v, cnt_hbm.at[w])


@functools.partial(
    pl.kernel,
    compiler_params=pltpu.CompilerParams(needs_layout_passes=False, use_tc_tiling_on_sc=False),
    out_type=(
        jax.ShapeDtypeStruct((EPAD,), _i32),
        jax.ShapeDtypeStruct((EPAD,), _i32),
        jax.ShapeDtypeStruct((BKT_PAD,), _i32),
        jax.ShapeDtypeStruct((BKT_PAD,), _i32),
    ),
    mesh=_mesh,
    scratch_types=[
        pltpu.VMEM((NWRK * BKT_PAD * 16,), _i32),
        pltpu.VMEM((BKT_PAD * 16,), _i32),
        pltpu.VMEM((BKT_PAD,), _i32),
        pltpu.VMEM((BKT_PAD,), _i32),
        pltpu.VMEM((CH,), _i32),
        pltpu.VMEM((CH,), _i32),
        pltpu.VMEM((CH,), _i32),
        pltpu.SemaphoreType.DMA,
    ],
)
def _k_place(dst_hbm, src_hbm, cntf_hbm, srcp_hbm, dstp_hbm, offp_hbm, deg_hbm,
             cnt_a, base16, deg_v, offp_v, srcv, dstv, posv, sem):
    w = _wid()
    pltpu.sync_copy(cntf_hbm, cnt_a)
    it16 = _iota16()
    zeros = jnp.zeros((16,), _i32)

    def degb(b, _):
        def accw(wi, s):
            return s + cnt_a[pl.ds((wi * BKT_PAD + b) * 16, 16)]

        tot = lax.fori_loop(0, NWRK, accw, zeros)
        s = jnp.sum(tot)
        plsc.store_scatter(deg_v, [_full(b)], _full(s), mask=it16 == 0)
        return 0

    lax.fori_loop(0, BKT_PAD, degb, 0)

    def pfx(gi, carry):
        d16 = deg_v[pl.ds(gi * 16, 16)]
        r16 = (d16 + 7) & _full(-8)
        cs = plsc.cumsum(r16)
        offp_v[pl.ds(gi * 16, 16)] = carry + cs - r16
        return carry + jnp.sum(r16)

    lax.fori_loop(0, BKT_PAD // 16, pfx, jnp.int32(0))

    @pl.when(w == 0)
    def _():
        pltpu.sync_copy(offp_v, offp_hbm)
        pltpu.sync_copy(deg_v, deg_hbm)

    def baseb(b, _):
        def accw(wi, s):
            row = cnt_a[pl.ds((wi * BKT_PAD + b) * 16, 16)]
            return s + jnp.where(wi < w, jnp.sum(row), 0)

        before = lax.fori_loop(0, NWRK, accw, jnp.int32(0))
        myrow = cnt_a[pl.ds((w * BKT_PAD + b) * 16, 16)]
        mycs = plsc.cumsum(myrow) - myrow
        ob = plsc.load_gather(offp_v, [_full(b)])
        base16[pl.ds(b * 16, 16)] = ob + before + mycs
        return 0

    lax.fori_loop(0, BKT_PAD, baseb, 0)

    dump = _full(EPAD - 16) + it16

    def chunk(ci, _):
        cb = pl.multiple_of(w * EWP + ci * CH, 128)
        pltpu.sync_copy(dst_hbm.at[pl.ds(cb, CH)], dstv)
        pltpu.sync_copy(src_hbm.at[pl.ds(cb, CH)], srcv)

        def place(g, _):
            d16 = dstv[pl.ds(g * 16, 16)]
            msk = (cb + g * 16 + it16) < E
            idx = (d16 >> 8) * 16 + it16
            p = plsc.load_gather(base16, [idx], mask=msk)
            plsc.store_scatter(base16, [idx], p + 1, mask=msk)
            posv[pl.ds(g * 16, 16)] = jnp.where(msk, p, dump)
            return 0

        lax.fori_loop(0, CH // 16, place, 0)
        pltpu.async_copy(srcv, srcp_hbm.at[posv], sem).wait()
        pltpu.async_copy(dstv, dstp_hbm.at[posv], sem).wait()
        return 0

    lax.fori_loop(0, EWP // CH, chunk, 0)


@functools.partial(
    pl.kernel,
    compiler_params=pltpu.CompilerParams(needs_layout_passes=False, use_tc_tiling_on_sc=False),
    out_type=jax.ShapeDtypeStruct((EPAD,), _f32),
    mesh=_mesh,
    scratch_types=[
        pltpu.VMEM((CH,), _i32),
        pltpu.VMEM((CH,), _i32),
        pltpu.VMEM((CH, 16), _f32),
        pltpu.VMEM((CH, 16), _f32),
        pltpu.VMEM((CH,), _f32),
        pltpu.SemaphoreType.DMA,
        pltpu.SemaphoreType.DMA,
    ],
)
def _k_dist(srcp_hbm, dstp_hbm, pos_hbm, dist_hbm, sv, dv, ps, pd, dout,
            sem1, sem2):
    w = _wid()
    it16 = _iota16()
    nmax = _full(N - 1)
    zeroi = jnp.zeros((16,), _i32)
    magic = _full(0x1FBD1DF5)
    half = _full(0.5, _f32)
    eps = _full(1e-12, _f32)

    def chunk(ci, _):
        e0 = pl.multiple_of(w * EDW + ci * CH, 128)
        pltpu.sync_copy(srcp_hbm.at[pl.ds(e0, CH)], sv)
        pltpu.sync_copy(dstp_hbm.at[pl.ds(e0, CH)], dv)

        def clampg(g, _):
            sv[pl.ds(g * 16, 16)] = jnp.minimum(
                jnp.maximum(sv[pl.ds(g * 16, 16)], zeroi), nmax)
            dv[pl.ds(g * 16, 16)] = jnp.minimum(
                jnp.maximum(dv[pl.ds(g * 16, 16)], zeroi), nmax)
            return 0

        lax.fori_loop(0, CH // 16, clampg, 0)
        pltpu.async_copy(pos_hbm.at[sv], ps, sem1).wait()
        pltpu.async_copy(pos_hbm.at[dv], pd, sem2).wait()

        def dot3(g, _):
            rows = g * 16 + it16
            s = eps
            for cdim in range(3):
                a = plsc.load_gather(ps, [rows, _full(cdim)])
                b = plsc.load_gather(pd, [rows, _full(cdim)])
                d = a - b
                s = s + d * d
            i = plsc.bitcast(s, _i32)
            y = plsc.bitcast(magic + (i >> 1), _f32)
            for _ in range(3):
                y = half * (y + s / y)
            dout[pl.ds(g * 16, 16)] = y
            return 0

        lax.fori_loop(0, CH // 16, dot3, 0)
        pltpu.sync_copy(dout, dist_hbm.at[pl.ds(e0, CH)])
        return 0

    lax.fori_loop(0, EDW // CH, chunk, 0)


_RBF_C = np.linspace(0.0, MAX_RADIUS, NB)
_RBF_I = 1.0 / (2.0 * (MAX_RADIUS / NB) ** 2)


@functools.partial(
    pl.kernel,
    compiler_params=pltpu.CompilerParams(needs_layout_passes=False, use_tc_tiling_on_sc=False),
    out_type=(
        jax.ShapeDtypeStruct((NPAD, DP), _f32),
        jax.ShapeDtypeStruct((NPAD, NRB), _f32),
    ),
    mesh=_mesh,
    scratch_types=[
        pltpu.VMEM((NPB, DP), _f32),
        pltpu.VMEM((NPB, NRB), _f32),
        pltpu.VMEM((2, CH, DP), _f32),
        pltpu.VMEM((2, CH, DP), _f32),
        pltpu.VMEM((CH, NRB), _f32),
        pltpu.VMEM((2, CH), _i32),
        pltpu.VMEM((2, CH), _i32),
        pltpu.VMEM((2, CH), _f32),
        pltpu.VMEM((2, CH), _i32),
        pltpu.VMEM((2, CH), _i32),
        pltpu.VMEM((2, CH), _i32),
        pltpu.VMEM((BKT_PAD,), _i32),
        pltpu.VMEM((BKT_PAD,), _i32),
        pltpu.VMEM((CH, DP), _f32),
        pltpu.VMEM_SHARED((16 * NPB, DP), _f32),
        pltpu.VMEM_SHARED((16 * NPB, NRB), _f32),
        pltpu.SemaphoreType.DMA,
        pltpu.SemaphoreType.DMA,
        pltpu.SemaphoreType.DMA,
        pltpu.SemaphoreType.DMA,
    ],
)
def _k_sweep(q_hbm, qr_hbm, kt_hbm, vt_hbm, srcp_hbm, dstp_hbm, dist_hbm,
             offp_hbm, deg_hbm, numv_hbm, numrd_hbm,
             q_l, qr_l, kc, vc, urd, srcv, dstv, distv, sidx, aidx, dlv,
             offp_v, deg_v, zblk, accv, accrd, semk, semv, sema, semb):
    w = _wid()
    sid = lax.axis_index("s")
    it16 = _iota16()
    pltpu.sync_copy(offp_hbm, offp_v)
    pltpu.sync_copy(deg_hbm, deg_v)

    zeros = jnp.zeros((16,), _f32)
    zeroi = jnp.zeros((16,), _i32)
    nmax = _full(N - 1)
    npbm1 = _full(NPB - 1)

    def zrow(r, _):
        for c in range(DP // 16):
            zblk[r, pl.ds(c * 16, 16)] = zeros
        return 0

    lax.fori_loop(0, CH, zrow, 0)

    for t in range(7):
        b = w + NWRK * t

        @pl.when(b < NBKT)
        def _():
            pltpu.sync_copy(q_hbm.at[pl.ds(pl.multiple_of(b * NPB, NPB), NPB)], q_l)
            pltpu.sync_copy(qr_hbm.at[pl.ds(pl.multiple_of(b * NPB, NPB), NPB)], qr_l)
            for z in range(NPB // CH):
                pltpu.sync_copy(zblk, accv.at[pl.ds(sid * NPB + z * CH, CH)])
                pltpu.sync_copy(zblk.at[:, pl.ds(0, NRB)],
                                accrd.at[pl.ds(sid * NPB + z * CH, CH)])

            degb = _elem(deg_v, b)
            offb = _elem(offp_v, b)
            nch = (degb + CH - 1) // CH

            def stage(ci, p):
                e0 = pl.multiple_of(offb + ci * CH, 8)
                pltpu.sync_copy(srcp_hbm.at[pl.ds(e0, CH)], srcv.at[p])
                pltpu.sync_copy(dstp_hbm.at[pl.ds(e0, CH)], dstv.at[p])
                pltpu.sync_copy(dist_hbm.at[pl.ds(e0, CH)], distv.at[p])

                def cleang(g, _):
                    s16 = srcv[p, pl.ds(g * 16, 16)]
                    d16 = dstv[p, pl.ds(g * 16, 16)]
                    sidx[p, pl.ds(g * 16, 16)] = jnp.minimum(
                        jnp.maximum(s16, zeroi), nmax)
                    dl = jnp.minimum(jnp.maximum(d16 - b * NPB, zeroi), npbm1)
                    dlv[p, pl.ds(g * 16, 16)] = dl
                    aidx[p, pl.ds(g * 16, 16)] = dl + sid * NPB
                    return 0

                lax.fori_loop(0, CH // 16, cleang, 0)
                pltpu.async_copy(kt_hbm.at[sidx.at[p]], kc.at[p], semk)
                pltpu.async_copy(vt_hbm.at[sidx.at[p]], vc.at[p], semv)

            @pl.when(nch > 0)
            def _():
                stage(jnp.int32(0), jnp.int32(0))

            def chunk(ci, _):
                p = lax.rem(ci, 2)

                @pl.when(ci + 1 < nch)
                def _():
                    stage(ci + 1, 1 - p)

                pltpu.make_async_copy(kt_hbm.at[pl.ds(0, CH)],
                                      kc.at[0], semk).wait()
                pltpu.make_async_copy(vt_hbm.at[pl.ds(0, CH)],
                                      vc.at[0], semv).wait()

                def logitg(g, _):
                    dlg = dlv[p, pl.ds(g * 16, 16)]
                    rows = g * 16 + it16
                    pp = _full(p)

                    def dotd(dd, accs):
                        a0, a1 = accs
                        qv0 = plsc.load_gather(q_l, [dlg, _full(2 * dd)])
                        kv0 = plsc.load_gather(kc, [pp, rows, _full(2 * dd)])
                        qv1 = plsc.load_gather(q_l, [dlg, _full(2 * dd + 1)])
                        kv1 = plsc.load_gather(kc, [pp, rows, _full(2 * dd + 1)])
                        return (a0 + qv0 * kv0, a1 + qv1 * kv1)

                    acc0, acc1 = lax.fori_loop(0, D // 2, dotd,
                                               (zeros, zeros), unroll=8)
                    acc = acc0 + acc1

                    dg = distv[p, pl.ds(g * 16, 16)]
                    acc2 = zeros
                    rjs = []
                    for j in range(NB):
                        dd = dg - _full(_RBF_C[j], _f32)
                        rj = jnp.exp(dd * dd * _full(-_RBF_I, _f32))
                        qrv = plsc.load_gather(qr_l, [dlg, _full(j)])
                        acc2 = acc2 + qrv * rj
                        rjs.append(rj)
                    lg = acc + acc2
                    valid = (ci * CH + g * 16 + it16) < degb
                    ex = jnp.where(valid, jnp.exp(lg), zeros)
                    for j in range(NB):
                        plsc.store_scatter(urd, [rows, _full(j)], rjs[j] * ex)
                    plsc.store_scatter(urd, [rows, _full(NB)], ex)
                    for j in range(NB + 1, NRB):
                        plsc.store_scatter(urd, [rows, _full(j)], zeros)

                    def upd(dd, exc):
                        vv0 = plsc.load_gather(vc, [pp, rows, _full(2 * dd)])
                        vv1 = plsc.load_gather(vc, [pp, rows, _full(2 * dd + 1)])
                        plsc.store_scatter(vc, [pp, rows, _full(2 * dd)],
                                           vv0 * exc)
                        plsc.store_scatter(vc, [pp, rows, _full(2 * dd + 1)],
                                           vv1 * exc)
                        return exc

                    lax.fori_loop(0, D // 2, upd, ex, unroll=8)
                    return 0

                lax.fori_loop(0, CH // 16, logitg, 0)
                ca = pltpu.async_copy(vc.at[p], accv.at[aidx.at[p]], sema,
                                      add=True)
                cb = pltpu.async_copy(urd, accrd.at[aidx.at[p]], semb,
                                      add=True)
                ca.wait()
                cb.wait()
                return 0

            lax.fori_loop(0, nch, chunk, 0)
            pltpu.sync_copy(accv.at[pl.ds(sid * NPB, NPB)],
                            numv_hbm.at[pl.ds(pl.multiple_of(b * NPB, NPB), NPB)])
            pltpu.sync_copy(accrd.at[pl.ds(sid * NPB, NPB)],
                            numrd_hbm.at[pl.ds(pl.multiple_of(b * NPB, NPB), NPB)])


def _tc_embed(xp, wembp):
    def body(x_ref, w_ref, o_ref):
        o_ref[...] = jnp.dot(x_ref[...], w_ref[...],
                             preferred_element_type=_f32)

    return pl.pallas_call(
        body,
        grid=(NPAD // RB,),
        in_specs=[
            pl.BlockSpec((RB, 8), lambda i: (i, 0)),
            pl.BlockSpec((8, DP), lambda i: (0, 0)),
        ],
        out_specs=pl.BlockSpec((RB, DP), lambda i: (i, 0)),
        out_shape=jax.ShapeDtypeStruct((NPAD, DP), _f32),
    )(xp, wembp)


def _tc_pre(h, aq, akh, avh, aqrt):
    def body(h_ref, aq_ref, akh_ref, avh_ref, aqrt_ref,
             q_ref, qr_ref, kn_ref, vn_ref):
        hb = h_ref[...]
        q = jnp.dot(hb, aq_ref[...], preferred_element_type=_f32) * SCALE
        q_ref[...] = q
        qr_ref[...] = jnp.dot(q, aqrt_ref[...], preferred_element_type=_f32)
        kn_ref[...] = jnp.dot(hb, akh_ref[...], preferred_element_type=_f32)
        vn_ref[...] = jnp.dot(hb, avh_ref[...], preferred_element_type=_f32)

    return pl.pallas_call(
        body,
        grid=(NPAD // RB,),
        in_specs=[
            pl.BlockSpec((RB, DP), lambda i: (i, 0)),
            pl.BlockSpec((DP, DP), lambda i: (0, 0)),
            pl.BlockSpec((DP, DP), lambda i: (0, 0)),
            pl.BlockSpec((DP, DP), lambda i: (0, 0)),
            pl.BlockSpec((DP, NRB), lambda i: (0, 0)),
        ],
        out_specs=[
            pl.BlockSpec((RB, DP), lambda i: (i, 0)),
            pl.BlockSpec((RB, NRB), lambda i: (i, 0)),
            pl.BlockSpec((RB, DP), lambda i: (i, 0)),
            pl.BlockSpec((RB, DP), lambda i: (i, 0)),
        ],
        out_shape=(
            jax.ShapeDtypeStruct((NPAD, DP), _f32),
            jax.ShapeDtypeStruct((NPAD, NRB), _f32),
            jax.ShapeDtypeStruct((NPAD, DP), _f32),
            jax.ShapeDtypeStruct((NPAD, DP), _f32),
        ),
    )(h, aq, akh, avh, aqrt)


def _tc_post(h, numv, numrd, wvrp, wop):
    def body(h_ref, nv_ref, nrd_ref, wvr_ref, wo_ref, o_ref):
        nrd = nrd_ref[...]
        num = nv_ref[...] + jnp.dot(nrd, wvr_ref[...],
                                    preferred_element_type=_f32)
        den = nrd[:, NB:NB + 1] + 1e-9
        agg = num / den
        t = h_ref[...] + jnp.dot(agg, wo_ref[...],
                                 preferred_element_type=_f32)
        s1 = jnp.sum(t, axis=-1, keepdims=True)
        s2 = jnp.sum(t * t, axis=-1, keepdims=True)
        mu = s1 / D
        var = s2 / D - mu * mu
        lnv = (t - mu) / jnp.sqrt(var + 1e-5)
        cmask = (lax.broadcasted_iota(_i32, (1, DP), 1) < D).astype(_f32)
        o_ref[...] = lnv * cmask

    return pl.pallas_call(
        body,
        grid=(NPAD // RB,),
        in_specs=[
            pl.BlockSpec((RB, DP), lambda i: (i, 0)),
            pl.BlockSpec((RB, DP), lambda i: (i, 0)),
            pl.BlockSpec((RB, NRB), lambda i: (i, 0)),
            pl.BlockSpec((NRB, DP), lambda i: (0, 0)),
            pl.BlockSpec((DP, DP), lambda i: (0, 0)),
        ],
        out_specs=pl.BlockSpec((RB, DP), lambda i: (i, 0)),
        out_shape=jax.ShapeDtypeStruct((NPAD, DP), _f32),
    )(h, numv, numrd, wvrp, wop)


def _tc_final(h, wwp):
    def body(h_ref, ww_ref, o_ref):
        hb = h_ref[...]
        lg = jnp.dot(hb, ww_ref[...], preferred_element_type=_f32)
        gate = 1.0 / (1.0 + jnp.exp(-lg[:, 0:1]))
        o_ref[...] = hb * gate

    return pl.pallas_call(
        body,
        grid=(NPAD // RB,),
        in_specs=[
            pl.BlockSpec((RB, DP), lambda i: (i, 0)),
            pl.BlockSpec((DP, NRB), lambda i: (0, 0)),
        ],
        out_specs=pl.BlockSpec((RB, DP), lambda i: (i, 0)),
        out_shape=jax.ShapeDtypeStruct((NPAD, DP), _f32),
    )(h, wwp)


def kernel(x, pos, edge_index, W_emb, Wq, Wk, Wv, Wo, Ww):
    src = edge_index[0].astype(_i32)
    dst = edge_index[1].astype(_i32)
    src_in = jnp.pad(src, (0, EIN - E))
    dst_in = jnp.pad(dst, (0, EIN - E))
    posp = jnp.pad(pos, ((0, 0), (0, 16 - 3)))
    xp = jnp.pad(x, ((0, NPAD - N), (0, 8 - D_IN)))
    wembp = jnp.pad(W_emb, ((0, 8 - D_IN), (0, DP - D)))

    cnt = _k_hist(dst_in)
    srcb, dstb, offp, deg = _k_place(dst_in, src_in, cnt.reshape(-1))
    dist = _k_dist(srcb, dstb, posp)

    h = _tc_embed(xp, wembp)
    for l in range(L):
        aq = jnp.pad(Wq[l], ((0, DP - D), (0, DP - D)))
        akh = jnp.pad(Wk[l][:D], ((0, DP - D), (0, DP - D)))
        avh = jnp.pad(Wv[l][:D], ((0, DP - D), (0, DP - D)))
        aqrt = jnp.pad(Wk[l][D:].T, ((0, DP - D), (0, NRB - NB)))
        wvrp = jnp.pad(Wv[l][D:], ((0, NRB - NB), (0, DP - D)))
        wop = jnp.pad(Wo[l], ((0, DP - D), (0, DP - D)))

        q, qr, kn, vn = _tc_pre(h, aq, akh, avh, aqrt)
        numv, numrd = _k_sweep(q, qr, kn, vn, srcb, dstb, dist, offp, deg)
        h = _tc_post(h, numv, numrd, wvrp, wop)

    wwp = jnp.pad(Ww, ((0, DP - D), (0, NRB - 1)))
    out = _tc_final(h, wwp)
    return out[:N, :D]
